# Initial kernel scaffold; baseline (speedup 1.0000x reference)
import functools

import jax
import jax.numpy as jnp
from jax import lax
from jax.experimental import pallas as pl
from jax.experimental.pallas import tpu as pltpu
from jax.experimental.pallas import tpu_sc as plsc

N = 10000
E = 320000
D = 128
NC = 2
NS = 16
NW = NC * NS
EPT = E // NW
CHUNK = 80
NCHUNK = EPT // CHUNK
RPT = N // NS
CW = 16


def _sc_pass(with_count: bool):
  mesh = plsc.VectorSubcoreMesh(core_axis_name="c", subcore_axis_name="s")

  def tile_loop(x_hbm, ei_hbm, out_hbm, src_v, dst_v, rows_v, acc_sh,
                cnt_parts=None):
    c = lax.axis_index("c")
    s = lax.axis_index("s")
    wid = c * NS + s

    def step(i, carry):
      base = wid * EPT + i * CHUNK
      pltpu.sync_copy(ei_hbm.at[0, pl.ds(base, CHUNK)], src_v)
      pltpu.sync_copy(ei_hbm.at[1, pl.ds(base, CHUNK)], dst_v)
      pltpu.sync_copy(x_hbm.at[src_v], rows_v)
      pltpu.sync_copy(rows_v, acc_sh.at[dst_v], add=True)
      if cnt_parts is not None:
        ones_v, cnt_sh, _ = cnt_parts
        pltpu.sync_copy(ones_v, cnt_sh.at[dst_v], add=True)
      return carry

    lax.fori_loop(0, NCHUNK, step, 0)
    plsc.subcore_barrier()
    rows = pl.ds(s * RPT, RPT)
    pltpu.sync_copy(acc_sh.at[rows], out_hbm.at[c, rows])
    if cnt_parts is not None:
      _, cnt_sh, cnt_hbm = cnt_parts
      pltpu.sync_copy(cnt_sh.at[rows], cnt_hbm.at[c, rows])

  if with_count:
    def body(x_hbm, ei_hbm, z128_hbm, z16_hbm, ones_hbm,
             out_hbm, cnt_hbm, src_v, dst_v, rows_v, ones_v, acc_sh, cnt_sh):
      s = lax.axis_index("s")
      rows = pl.ds(s * RPT, RPT)
      pltpu.sync_copy(z128_hbm, acc_sh.at[rows])
      pltpu.sync_copy(z16_hbm, cnt_sh.at[rows])
      pltpu.sync_copy(ones_hbm, ones_v)
      plsc.subcore_barrier()
      tile_loop(x_hbm, ei_hbm, out_hbm, src_v, dst_v, rows_v, acc_sh,
                (ones_v, cnt_sh, cnt_hbm))

    out_type = [jax.ShapeDtypeStruct((NC, N, D), jnp.float32),
                jax.ShapeDtypeStruct((NC, N, CW), jnp.float32)]
    scratch = [pltpu.VMEM((CHUNK,), jnp.int32),
               pltpu.VMEM((CHUNK,), jnp.int32),
               pltpu.VMEM((CHUNK, D), jnp.float32),
               pltpu.VMEM((CHUNK, CW), jnp.float32),
               pltpu.VMEM_SHARED((N, D), jnp.float32),
               pltpu.VMEM_SHARED((N, CW), jnp.float32)]
  else:
    def body(x_hbm, ei_hbm, z128_hbm, out_hbm, src_v, dst_v, rows_v, acc_sh):
      s = lax.axis_index("s")
      pltpu.sync_copy(z128_hbm, acc_sh.at[pl.ds(s * RPT, RPT)])
      plsc.subcore_barrier()
      tile_loop(x_hbm, ei_hbm, out_hbm, src_v, dst_v, rows_v, acc_sh)

    out_type = [jax.ShapeDtypeStruct((NC, N, D), jnp.float32)]
    scratch = [pltpu.VMEM((CHUNK,), jnp.int32),
               pltpu.VMEM((CHUNK,), jnp.int32),
               pltpu.VMEM((CHUNK, D), jnp.float32),
               pltpu.VMEM_SHARED((N, D), jnp.float32)]

  return pl.kernel(body, out_type=out_type, mesh=mesh, scratch_types=scratch)


_sc_agg_count = _sc_pass(with_count=True)
_sc_agg = _sc_pass(with_count=False)


def _combine(p, cnt, x, wl_t, bl, wr_t, relu):
  BR = 1000

  def body(p_ref, c_ref, x_ref, wl_ref, bl_ref, wr_ref, o_ref):
    ssum = p_ref[0] + p_ref[1]
    cc = c_ref[0] + c_ref[1]
    mean = ssum * (1.0 / jnp.maximum(cc[:, :1], 1.0))
    y = jnp.dot(mean, wl_ref[...], preferred_element_type=jnp.float32)
    y = y + jnp.dot(x_ref[...], wr_ref[...], preferred_element_type=jnp.float32)
    y = y + bl_ref[...]
    if relu:
      y = jnp.maximum(y, 0.0)
    o_ref[...] = y

  return pl.pallas_call(
      body,
      grid=(N // BR,),
      in_specs=[
          pl.BlockSpec((NC, BR, D), lambda i: (0, i, 0)),
          pl.BlockSpec((NC, BR, CW), lambda i: (0, i, 0)),
          pl.BlockSpec((BR, D), lambda i: (i, 0)),
          pl.BlockSpec((D, D), lambda i: (0, 0)),
          pl.BlockSpec((1, D), lambda i: (0, 0)),
          pl.BlockSpec((D, D), lambda i: (0, 0)),
      ],
      out_specs=pl.BlockSpec((BR, D), lambda i: (i, 0)),
      out_shape=jax.ShapeDtypeStruct((N, D), jnp.float32),
  )(p, cnt, x, wl_t, bl, wr_t)


def kernel(x, edge_index, W1l, b1l, W1r, W2l, b2l, W2r):
  ei = edge_index.astype(jnp.int32)
  z128 = jnp.zeros((RPT, D), jnp.float32)
  z16 = jnp.zeros((RPT, CW), jnp.float32)
  ones = jnp.ones((CHUNK, CW), jnp.float32)

  p1, cnt = _sc_agg_count(x, ei, z128, z16, ones)
  h = _combine(p1, cnt, x, W1l.T, b1l.reshape(1, D), W1r.T, relu=True)
  p2, = _sc_agg(h, ei, z128)
  out = _combine(p2, cnt, h, W2l.T, b2l.reshape(1, D), W2r.T, relu=False)
  return out


# speedup vs baseline: 3.3593x; 3.3593x over previous
import jax
import jax.numpy as jnp
from jax import lax
from jax.experimental import pallas as pl
from jax.experimental.pallas import tpu as pltpu
from jax.experimental.pallas import tpu_sc as plsc

N = 10000
E = 320000
D = 128
HD = D // 2
NC = 2
NS = 16
NW = NC * NS
EPT = E // NW
CHUNK = 80
NCHUNK = EPT // CHUNK
ROWS0 = 624
LAST = N - ROWS0 * (NS - 1)
CW = 16


def _per_tile_rows(s, fn):
  @pl.when(s < NS - 1)
  def _():
    fn(pl.multiple_of(s * ROWS0, 8), ROWS0)

  @pl.when(s == NS - 1)
  def _():
    fn((NS - 1) * ROWS0, LAST)


def _sc_pass(with_count: bool):
  mesh = plsc.VectorSubcoreMesh(core_axis_name="c", subcore_axis_name="s")

  def phase(c, s, x_hbm, src_hbm, dst_hbm, out_hbm, z64_hbm,
            src_v, dst_v, rows_v, stage_v, acc_sh, sem, cnt_parts=None):
    wid = c * NS + s

    pltpu.sync_copy(z64_hbm, stage_v)

    def zero(off, nrows):
      pltpu.sync_copy(stage_v.at[pl.ds(0, nrows)], acc_sh.at[pl.ds(off, nrows)])
      if cnt_parts is not None:
        _, cnt_sh, _, cstage_v, z16_hbm = cnt_parts
        pltpu.sync_copy(cstage_v.at[pl.ds(0, nrows)],
                        cnt_sh.at[pl.ds(off, nrows)])

    if cnt_parts is not None:
      _, _, _, cstage_v, z16_hbm = cnt_parts
      pltpu.sync_copy(z16_hbm, cstage_v)
    _per_tile_rows(s, zero)
    plsc.subcore_barrier()

    def step(i, carry):
      base = pl.multiple_of(wid * EPT + i * CHUNK, 8)
      pltpu.sync_copy(src_hbm.at[pl.ds(base, CHUNK)], src_v)
      pltpu.sync_copy(dst_hbm.at[pl.ds(base, CHUNK)], dst_v)
      pltpu.async_copy(x_hbm.at[src_v], rows_v, sem).wait()
      pltpu.sync_copy(rows_v, acc_sh.at[dst_v], add=True)
      if cnt_parts is not None:
        ones_v, cnt_sh, _, _, _ = cnt_parts
        pltpu.sync_copy(ones_v, cnt_sh.at[dst_v], add=True)
      return carry

    lax.fori_loop(0, NCHUNK, step, 0)
    plsc.subcore_barrier()

    def copy_out(off, nrows):
      rows = pl.ds(off, nrows)
      pltpu.sync_copy(acc_sh.at[rows], stage_v.at[pl.ds(0, nrows)])
      pltpu.sync_copy(stage_v.at[pl.ds(0, nrows)], out_hbm.at[c, rows])
      if cnt_parts is not None:
        _, cnt_sh, cnt_hbm, cstage_v, _ = cnt_parts
        pltpu.sync_copy(cnt_sh.at[rows], cstage_v.at[pl.ds(0, nrows)])
        pltpu.sync_copy(cstage_v.at[pl.ds(0, nrows)], cnt_hbm.at[c, rows])

    _per_tile_rows(s, copy_out)

  if with_count:
    def body(xa_hbm, xb_hbm, src_hbm, dst_hbm, z64_hbm, z16_hbm, ones_hbm,
             pa_hbm, pb_hbm, cnt_hbm,
             src_v, dst_v, rows_v, ones_v, stage_v, cstage_v,
             acc_sh, cnt_sh, sem):
      c = lax.axis_index("c")
      s = lax.axis_index("s")
      pltpu.sync_copy(ones_hbm, ones_v)
      phase(c, s, xa_hbm, src_hbm, dst_hbm, pa_hbm, z64_hbm,
            src_v, dst_v, rows_v, stage_v, acc_sh, sem,
            (ones_v, cnt_sh, cnt_hbm, cstage_v, z16_hbm))
      phase(c, s, xb_hbm, src_hbm, dst_hbm, pb_hbm, z64_hbm,
            src_v, dst_v, rows_v, stage_v, acc_sh, sem)

    out_type = [jax.ShapeDtypeStruct((NC, N, HD), jnp.float32),
                jax.ShapeDtypeStruct((NC, N, HD), jnp.float32),
                jax.ShapeDtypeStruct((NC, N, CW), jnp.float32)]
    scratch = [pltpu.VMEM((CHUNK,), jnp.int32),
               pltpu.VMEM((CHUNK,), jnp.int32),
               pltpu.VMEM((CHUNK, HD), jnp.float32),
               pltpu.VMEM((CHUNK, CW), jnp.float32),
               pltpu.VMEM((LAST, HD), jnp.float32),
               pltpu.VMEM((LAST, CW), jnp.float32),
               pltpu.VMEM_SHARED((N, HD), jnp.float32),
               pltpu.VMEM_SHARED((N, CW), jnp.float32),
               pltpu.SemaphoreType.DMA]
  else:
    def body(xa_hbm, xb_hbm, src_hbm, dst_hbm, z64_hbm,
             pa_hbm, pb_hbm,
             src_v, dst_v, rows_v, stage_v, acc_sh, sem):
      c = lax.axis_index("c")
      s = lax.axis_index("s")
      phase(c, s, xa_hbm, src_hbm, dst_hbm, pa_hbm, z64_hbm,
            src_v, dst_v, rows_v, stage_v, acc_sh, sem)
      phase(c, s, xb_hbm, src_hbm, dst_hbm, pb_hbm, z64_hbm,
            src_v, dst_v, rows_v, stage_v, acc_sh, sem)

    out_type = [jax.ShapeDtypeStruct((NC, N, HD), jnp.float32),
                jax.ShapeDtypeStruct((NC, N, HD), jnp.float32)]
    scratch = [pltpu.VMEM((CHUNK,), jnp.int32),
               pltpu.VMEM((CHUNK,), jnp.int32),
               pltpu.VMEM((CHUNK, HD), jnp.float32),
               pltpu.VMEM((LAST, HD), jnp.float32),
               pltpu.VMEM_SHARED((N, HD), jnp.float32),
               pltpu.SemaphoreType.DMA]

  return pl.kernel(body, out_type=out_type, mesh=mesh, scratch_types=scratch,
                   compiler_params=pltpu.CompilerParams(
                       use_tc_tiling_on_sc=False))


_sc_agg_count = _sc_pass(with_count=True)
_sc_agg = _sc_pass(with_count=False)


def _combine(pa, pb, cnt, x, wl_t, bl, wr_t, relu):
  BR = 1000

  def body(pa_ref, pb_ref, c_ref, x_ref, wl_ref, bl_ref, wr_ref, o_ref):
    cc = c_ref[0] + c_ref[1]
    inv = 1.0 / jnp.maximum(cc[:, :1], 1.0)
    mean_a = (pa_ref[0] + pa_ref[1]) * inv
    mean_b = (pb_ref[0] + pb_ref[1]) * inv
    y = jnp.dot(mean_a, wl_ref[:HD, :], preferred_element_type=jnp.float32)
    y = y + jnp.dot(mean_b, wl_ref[HD:, :], preferred_element_type=jnp.float32)
    y = y + jnp.dot(x_ref[...], wr_ref[...], preferred_element_type=jnp.float32)
    y = y + bl_ref[...]
    if relu:
      y = jnp.maximum(y, 0.0)
    o_ref[...] = y

  return pl.pallas_call(
      body,
      grid=(N // BR,),
      in_specs=[
          pl.BlockSpec((NC, BR, HD), lambda i: (0, i, 0)),
          pl.BlockSpec((NC, BR, HD), lambda i: (0, i, 0)),
          pl.BlockSpec((NC, BR, CW), lambda i: (0, i, 0)),
          pl.BlockSpec((BR, D), lambda i: (i, 0)),
          pl.BlockSpec((D, D), lambda i: (0, 0)),
          pl.BlockSpec((1, D), lambda i: (0, 0)),
          pl.BlockSpec((D, D), lambda i: (0, 0)),
      ],
      out_specs=pl.BlockSpec((BR, D), lambda i: (i, 0)),
      out_shape=jax.ShapeDtypeStruct((N, D), jnp.float32),
  )(pa, pb, cnt, x, wl_t, bl, wr_t)


def kernel(x, edge_index, W1l, b1l, W1r, W2l, b2l, W2r):
  ei = edge_index.astype(jnp.int32)
  src_idx = ei[0]
  dst_idx = ei[1]
  z64 = jnp.zeros((LAST, HD), jnp.float32)
  z16 = jnp.zeros((LAST, CW), jnp.float32)
  ones = jnp.ones((CHUNK, CW), jnp.float32)

  xa = x[:, :HD]
  xb = x[:, HD:]
  pa1, pb1, cnt = _sc_agg_count(xa, xb, src_idx, dst_idx, z64, z16, ones)
  h = _combine(pa1, pb1, cnt, x, W1l.T, b1l.reshape(1, D), W1r.T, relu=True)
  pa2, pb2 = _sc_agg(h[:, :HD], h[:, HD:], src_idx, dst_idx, z64)
  out = _combine(pa2, pb2, cnt, h, W2l.T, b2l.reshape(1, D), W2r.T, relu=False)
  return out


# speedup vs baseline: 8.3987x; 2.5001x over previous
import jax
import jax.numpy as jnp
from jax import lax
from jax.experimental import pallas as pl
from jax.experimental.pallas import tpu as pltpu
from jax.experimental.pallas import tpu_sc as plsc

N = 10000
E = 320000
D = 128
HD = D // 2
NC = 2
NS = 16
NW = NC * NS
EPT = E // NW
CHUNK = 80
NCHUNK = EPT // CHUNK
NPAIR = (NCHUNK - 1) // 2
ROWS0 = 624
LAST = N - ROWS0 * (NS - 1)
CW = 16
STG = 320


def _sub_chunks(nrows):
  out, o = [], 0
  while o < nrows:
    n = min(STG, nrows - o)
    out.append((o, n))
    o += n
  return out


def _per_tile_rows(s, fn):
  @pl.when(s < NS - 1)
  def _():
    fn(pl.multiple_of(s * ROWS0, 8), ROWS0)

  @pl.when(s == NS - 1)
  def _():
    fn((NS - 1) * ROWS0, LAST)


def _sc_pass(with_count: bool):
  mesh = plsc.VectorSubcoreMesh(core_axis_name="c", subcore_axis_name="s")

  def phase(c, s, x_hbm, out_hbm, z64_hbm, src_v, dst_v,
            rows0_v, rows1_v, stage_v, acc_sh, sem0, sem1, cnt_parts=None):
    pltpu.sync_copy(z64_hbm, stage_v)

    def zero(off, nrows):
      for so, sn in _sub_chunks(nrows):
        pltpu.sync_copy(stage_v.at[pl.ds(0, sn)],
                        acc_sh.at[pl.ds(off + so, sn)])
        if cnt_parts is not None:
          _, cnt_sh, _, cstage_v, z16_hbm = cnt_parts
          pltpu.sync_copy(cstage_v.at[pl.ds(0, sn)],
                          cnt_sh.at[pl.ds(off + so, sn)])

    if cnt_parts is not None:
      _, _, _, cstage_v, z16_hbm = cnt_parts
      pltpu.sync_copy(z16_hbm, cstage_v)
    _per_tile_rows(s, zero)
    plsc.subcore_barrier()

    def gather(i, buf, sem):
      pltpu.async_copy(x_hbm.at[src_v.at[i]], buf, sem)

    def drain(i, buf, sem):
      pltpu.make_async_copy(x_hbm.at[src_v.at[i]], buf, sem).wait()

    def scatter(i, buf):
      pltpu.sync_copy(buf, acc_sh.at[dst_v.at[i]], add=True)
      if cnt_parts is not None:
        ones_v, cnt_sh, _, _, _ = cnt_parts
        pltpu.sync_copy(ones_v, cnt_sh.at[dst_v.at[i]], add=True)

    gather(0, rows0_v, sem0)

    def pair(i2, carry):
      i = i2 * 2
      gather(i + 1, rows1_v, sem1)
      drain(i, rows0_v, sem0)
      scatter(i, rows0_v)
      gather(i + 2, rows0_v, sem0)
      drain(i + 1, rows1_v, sem1)
      scatter(i + 1, rows1_v)
      return carry

    lax.fori_loop(0, NPAIR, pair, 0)
    drain(NCHUNK - 1, rows0_v, sem0)
    scatter(NCHUNK - 1, rows0_v)
    plsc.subcore_barrier()

    def copy_out(off, nrows):
      for so, sn in _sub_chunks(nrows):
        rows = pl.ds(off + so, sn)
        pltpu.sync_copy(acc_sh.at[rows], stage_v.at[pl.ds(0, sn)])
        pltpu.sync_copy(stage_v.at[pl.ds(0, sn)], out_hbm.at[c, rows])
        if cnt_parts is not None:
          _, cnt_sh, cnt_hbm, cstage_v, _ = cnt_parts
          pltpu.sync_copy(cnt_sh.at[rows], cstage_v.at[pl.ds(0, sn)])
          pltpu.sync_copy(cstage_v.at[pl.ds(0, sn)], cnt_hbm.at[c, rows])

    _per_tile_rows(s, copy_out)

  if with_count:
    def body(xa_hbm, xb_hbm, src_hbm, dst_hbm, z64_hbm, z16_hbm, ones_hbm,
             pa_hbm, pb_hbm, cnt_hbm,
             src_v, dst_v, rows0_v, rows1_v, ones_v, stage_v, cstage_v,
             acc_sh, cnt_sh, sem0, sem1):
      c = lax.axis_index("c")
      s = lax.axis_index("s")
      wid = c * NS + s
      pltpu.sync_copy(src_hbm.at[wid], src_v)
      pltpu.sync_copy(dst_hbm.at[wid], dst_v)
      pltpu.sync_copy(ones_hbm, ones_v)
      phase(c, s, xa_hbm, pa_hbm, z64_hbm, src_v, dst_v,
            rows0_v, rows1_v, stage_v, acc_sh, sem0, sem1,
            (ones_v, cnt_sh, cnt_hbm, cstage_v, z16_hbm))
      phase(c, s, xb_hbm, pb_hbm, z64_hbm, src_v, dst_v,
            rows0_v, rows1_v, stage_v, acc_sh, sem0, sem1)

    out_type = [jax.ShapeDtypeStruct((NC, N, HD), jnp.float32),
                jax.ShapeDtypeStruct((NC, N, HD), jnp.float32),
                jax.ShapeDtypeStruct((NC, N, CW), jnp.float32)]
    scratch = [pltpu.VMEM((NCHUNK, CHUNK), jnp.int32),
               pltpu.VMEM((NCHUNK, CHUNK), jnp.int32),
               pltpu.VMEM((CHUNK, HD), jnp.float32),
               pltpu.VMEM((CHUNK, HD), jnp.float32),
               pltpu.VMEM((CHUNK, CW), jnp.float32),
               pltpu.VMEM((STG, HD), jnp.float32),
               pltpu.VMEM((STG, CW), jnp.float32),
               pltpu.VMEM_SHARED((N, HD), jnp.float32),
               pltpu.VMEM_SHARED((N, CW), jnp.float32),
               pltpu.SemaphoreType.DMA,
               pltpu.SemaphoreType.DMA]
  else:
    def body(xa_hbm, xb_hbm, src_hbm, dst_hbm, z64_hbm,
             pa_hbm, pb_hbm,
             src_v, dst_v, rows0_v, rows1_v, stage_v, acc_sh, sem0, sem1):
      c = lax.axis_index("c")
      s = lax.axis_index("s")
      wid = c * NS + s
      pltpu.sync_copy(src_hbm.at[wid], src_v)
      pltpu.sync_copy(dst_hbm.at[wid], dst_v)
      phase(c, s, xa_hbm, pa_hbm, z64_hbm, src_v, dst_v,
            rows0_v, rows1_v, stage_v, acc_sh, sem0, sem1)
      phase(c, s, xb_hbm, pb_hbm, z64_hbm, src_v, dst_v,
            rows0_v, rows1_v, stage_v, acc_sh, sem0, sem1)

    out_type = [jax.ShapeDtypeStruct((NC, N, HD), jnp.float32),
                jax.ShapeDtypeStruct((NC, N, HD), jnp.float32)]
    scratch = [pltpu.VMEM((NCHUNK, CHUNK), jnp.int32),
               pltpu.VMEM((NCHUNK, CHUNK), jnp.int32),
               pltpu.VMEM((CHUNK, HD), jnp.float32),
               pltpu.VMEM((CHUNK, HD), jnp.float32),
               pltpu.VMEM((STG, HD), jnp.float32),
               pltpu.VMEM_SHARED((N, HD), jnp.float32),
               pltpu.SemaphoreType.DMA,
               pltpu.SemaphoreType.DMA]

  return pl.kernel(body, out_type=out_type, mesh=mesh, scratch_types=scratch,
                   compiler_params=pltpu.CompilerParams(
                       use_tc_tiling_on_sc=False))


_sc_agg_count = _sc_pass(with_count=True)
_sc_agg = _sc_pass(with_count=False)


def _combine(pa, pb, cnt, x, wl_t, bl, wr_t, relu):
  BR = 1000

  def body(pa_ref, pb_ref, c_ref, x_ref, wl_ref, bl_ref, wr_ref, o_ref):
    cc = c_ref[0] + c_ref[1]
    inv = 1.0 / jnp.maximum(cc[:, :1], 1.0)
    mean_a = (pa_ref[0] + pa_ref[1]) * inv
    mean_b = (pb_ref[0] + pb_ref[1]) * inv
    y = jnp.dot(mean_a, wl_ref[:HD, :], preferred_element_type=jnp.float32)
    y = y + jnp.dot(mean_b, wl_ref[HD:, :], preferred_element_type=jnp.float32)
    y = y + jnp.dot(x_ref[...], wr_ref[...], preferred_element_type=jnp.float32)
    y = y + bl_ref[...]
    if relu:
      y = jnp.maximum(y, 0.0)
    o_ref[...] = y

  return pl.pallas_call(
      body,
      grid=(N // BR,),
      in_specs=[
          pl.BlockSpec((NC, BR, HD), lambda i: (0, i, 0)),
          pl.BlockSpec((NC, BR, HD), lambda i: (0, i, 0)),
          pl.BlockSpec((NC, BR, CW), lambda i: (0, i, 0)),
          pl.BlockSpec((BR, D), lambda i: (i, 0)),
          pl.BlockSpec((D, D), lambda i: (0, 0)),
          pl.BlockSpec((1, D), lambda i: (0, 0)),
          pl.BlockSpec((D, D), lambda i: (0, 0)),
      ],
      out_specs=pl.BlockSpec((BR, D), lambda i: (i, 0)),
      out_shape=jax.ShapeDtypeStruct((N, D), jnp.float32),
  )(pa, pb, cnt, x, wl_t, bl, wr_t)


def kernel(x, edge_index, W1l, b1l, W1r, W2l, b2l, W2r):
  ei = edge_index.astype(jnp.int32)
  src_idx = ei[0].reshape(NW, NCHUNK, CHUNK)
  dst_idx = ei[1].reshape(NW, NCHUNK, CHUNK)
  z64 = jnp.zeros((STG, HD), jnp.float32)
  z16 = jnp.zeros((STG, CW), jnp.float32)
  ones = jnp.ones((CHUNK, CW), jnp.float32)

  xa = x[:, :HD]
  xb = x[:, HD:]
  pa1, pb1, cnt = _sc_agg_count(xa, xb, src_idx, dst_idx, z64, z16, ones)
  h = _combine(pa1, pb1, cnt, x, W1l.T, b1l.reshape(1, D), W1r.T, relu=True)
  pa2, pb2 = _sc_agg(h[:, :HD], h[:, HD:], src_idx, dst_idx, z64)
  out = _combine(pa2, pb2, cnt, h, W2l.T, b2l.reshape(1, D), W2r.T, relu=False)
  return out


# speedup vs baseline: 11.1965x; 1.3331x over previous
import jax
import jax.numpy as jnp
from jax import lax
from jax.experimental import pallas as pl
from jax.experimental.pallas import tpu as pltpu
from jax.experimental.pallas import tpu_sc as plsc

N = 10000
E = 320000
D = 128
HD = D // 2
NC = 2
NS = 16
NW = NC * NS
EPT = E // NW
CHUNK = 80
NCHUNK = EPT // CHUNK
NB = 5
LOOK = 3
NGRP = NCHUNK // NB
ROWS0 = 624
LAST = N - ROWS0 * (NS - 1)
CW = 16
STG = 320


def _sub_chunks(nrows):
  out, o = [], 0
  while o < nrows:
    n = min(STG, nrows - o)
    out.append((o, n))
    o += n
  return out


def _per_tile_rows(s, fn):
  @pl.when(s < NS - 1)
  def _():
    fn(pl.multiple_of(s * ROWS0, 8), ROWS0)

  @pl.when(s == NS - 1)
  def _():
    fn((NS - 1) * ROWS0, LAST)


def _sc_pass(with_count: bool):
  mesh = plsc.VectorSubcoreMesh(core_axis_name="c", subcore_axis_name="s")

  def phase(c, s, x_hbm, out_hbm, z64_hbm, src_v, dst_v,
            rows, stage_v, acc_sh, gsem, ssem, cnt_parts=None):
    pltpu.sync_copy(z64_hbm, stage_v)

    def zero(off, nrows):
      for so, sn in _sub_chunks(nrows):
        pltpu.sync_copy(stage_v.at[pl.ds(0, sn)],
                        acc_sh.at[pl.ds(off + so, sn)])
        if cnt_parts is not None:
          _, cnt_sh, _, cstage_v, z16_hbm, _ = cnt_parts
          pltpu.sync_copy(cstage_v.at[pl.ds(0, sn)],
                          cnt_sh.at[pl.ds(off + so, sn)])

    if cnt_parts is not None:
      _, _, _, cstage_v, z16_hbm, _ = cnt_parts
      pltpu.sync_copy(z16_hbm, cstage_v)
    _per_tile_rows(s, zero)
    plsc.subcore_barrier()

    def gstart(i, j):
      pltpu.async_copy(x_hbm.at[src_v.at[i]], rows[j], gsem[j])

    def gwait(i, j):
      pltpu.make_async_copy(x_hbm.at[src_v.at[i]], rows[j], gsem[j]).wait()

    def sstart(i, j):
      pltpu.async_copy(rows[j], acc_sh.at[dst_v.at[i]], ssem[j], add=True)
      if cnt_parts is not None:
        ones_v, cnt_sh, _, _, _, csem = cnt_parts
        pltpu.async_copy(ones_v, cnt_sh.at[dst_v.at[i]], csem[j], add=True)

    def swait(i, j):
      pltpu.make_async_copy(rows[j], acc_sh.at[dst_v.at[i]], ssem[j]).wait()
      if cnt_parts is not None:
        ones_v, cnt_sh, _, _, _, csem = cnt_parts
        pltpu.make_async_copy(ones_v, cnt_sh.at[dst_v.at[i]], csem[j]).wait()

    for j in range(NB):
      gstart(j, j)
      m = j - LOOK
      if m >= 0:
        gwait(m, m)
        sstart(m, m)

    def group(g, carry):
      for j in range(NB):
        i = g * NB + j
        swait(i - NB, j)
        gstart(i, j)
        jm = (j + NB - LOOK) % NB
        gwait(i - LOOK, jm)
        sstart(i - LOOK, jm)
      return carry

    lax.fori_loop(1, NGRP, group, 0)

    for m in range(NCHUNK - LOOK, NCHUNK):
      j = m % NB
      gwait(m, j)
      sstart(m, j)
    for m in range(NCHUNK - NB, NCHUNK):
      swait(m, m % NB)
    plsc.subcore_barrier()

    def copy_out(off, nrows):
      for so, sn in _sub_chunks(nrows):
        rws = pl.ds(off + so, sn)
        pltpu.sync_copy(acc_sh.at[rws], stage_v.at[pl.ds(0, sn)])
        pltpu.sync_copy(stage_v.at[pl.ds(0, sn)], out_hbm.at[c, rws])
        if cnt_parts is not None:
          _, cnt_sh, cnt_hbm, cstage_v, _, _ = cnt_parts
          pltpu.sync_copy(cnt_sh.at[rws], cstage_v.at[pl.ds(0, sn)])
          pltpu.sync_copy(cstage_v.at[pl.ds(0, sn)], cnt_hbm.at[c, rws])

    _per_tile_rows(s, copy_out)

  if with_count:
    def body(xa_hbm, xb_hbm, src_hbm, dst_hbm, z64_hbm, z16_hbm, ones_hbm,
             pa_hbm, pb_hbm, cnt_hbm, *sc):
      src_v, dst_v = sc[0], sc[1]
      rows = list(sc[2:2 + NB])
      ones_v, stage_v, cstage_v, acc_sh, cnt_sh = sc[2 + NB:7 + NB]
      gsem = list(sc[7 + NB:7 + 2 * NB])
      ssem = list(sc[7 + 2 * NB:7 + 3 * NB])
      csem = list(sc[7 + 3 * NB:7 + 4 * NB])
      c = lax.axis_index("c")
      s = lax.axis_index("s")
      wid = c * NS + s
      pltpu.sync_copy(src_hbm.at[wid], src_v)
      pltpu.sync_copy(dst_hbm.at[wid], dst_v)
      pltpu.sync_copy(ones_hbm, ones_v)
      phase(c, s, xa_hbm, pa_hbm, z64_hbm, src_v, dst_v,
            rows, stage_v, acc_sh, gsem, ssem,
            (ones_v, cnt_sh, cnt_hbm, cstage_v, z16_hbm, csem))
      phase(c, s, xb_hbm, pb_hbm, z64_hbm, src_v, dst_v,
            rows, stage_v, acc_sh, gsem, ssem)

    out_type = [jax.ShapeDtypeStruct((NC, N, HD), jnp.float32),
                jax.ShapeDtypeStruct((NC, N, HD), jnp.float32),
                jax.ShapeDtypeStruct((NC, N, CW), jnp.float32)]
    scratch = ([pltpu.VMEM((NCHUNK, CHUNK), jnp.int32),
                pltpu.VMEM((NCHUNK, CHUNK), jnp.int32)]
               + [pltpu.VMEM((CHUNK, HD), jnp.float32) for _ in range(NB)]
               + [pltpu.VMEM((CHUNK, CW), jnp.float32),
                  pltpu.VMEM((STG, HD), jnp.float32),
                  pltpu.VMEM((STG, CW), jnp.float32),
                  pltpu.VMEM_SHARED((N, HD), jnp.float32),
                  pltpu.VMEM_SHARED((N, CW), jnp.float32)]
               + [pltpu.SemaphoreType.DMA] * (3 * NB))
  else:
    def body(xa_hbm, xb_hbm, src_hbm, dst_hbm, z64_hbm,
             pa_hbm, pb_hbm, *sc):
      src_v, dst_v = sc[0], sc[1]
      rows = list(sc[2:2 + NB])
      stage_v, acc_sh = sc[2 + NB], sc[3 + NB]
      gsem = list(sc[4 + NB:4 + 2 * NB])
      ssem = list(sc[4 + 2 * NB:4 + 3 * NB])
      c = lax.axis_index("c")
      s = lax.axis_index("s")
      wid = c * NS + s
      pltpu.sync_copy(src_hbm.at[wid], src_v)
      pltpu.sync_copy(dst_hbm.at[wid], dst_v)
      phase(c, s, xa_hbm, pa_hbm, z64_hbm, src_v, dst_v,
            rows, stage_v, acc_sh, gsem, ssem)
      phase(c, s, xb_hbm, pb_hbm, z64_hbm, src_v, dst_v,
            rows, stage_v, acc_sh, gsem, ssem)

    out_type = [jax.ShapeDtypeStruct((NC, N, HD), jnp.float32),
                jax.ShapeDtypeStruct((NC, N, HD), jnp.float32)]
    scratch = ([pltpu.VMEM((NCHUNK, CHUNK), jnp.int32),
                pltpu.VMEM((NCHUNK, CHUNK), jnp.int32)]
               + [pltpu.VMEM((CHUNK, HD), jnp.float32) for _ in range(NB)]
               + [pltpu.VMEM((STG, HD), jnp.float32),
                  pltpu.VMEM_SHARED((N, HD), jnp.float32)]
               + [pltpu.SemaphoreType.DMA] * (2 * NB))

  return pl.kernel(body, out_type=out_type, mesh=mesh, scratch_types=scratch,
                   compiler_params=pltpu.CompilerParams(
                       use_tc_tiling_on_sc=False))


_sc_agg_count = _sc_pass(with_count=True)
_sc_agg = _sc_pass(with_count=False)


def _combine(pa, pb, cnt, x, wl_t, bl, wr_t, relu):
  BR = 1000

  def body(pa_ref, pb_ref, c_ref, x_ref, wl_ref, bl_ref, wr_ref, o_ref):
    cc = c_ref[0] + c_ref[1]
    inv = 1.0 / jnp.maximum(cc[:, :1], 1.0)
    mean_a = (pa_ref[0] + pa_ref[1]) * inv
    mean_b = (pb_ref[0] + pb_ref[1]) * inv
    y = jnp.dot(mean_a, wl_ref[:HD, :], preferred_element_type=jnp.float32)
    y = y + jnp.dot(mean_b, wl_ref[HD:, :], preferred_element_type=jnp.float32)
    y = y + jnp.dot(x_ref[...], wr_ref[...], preferred_element_type=jnp.float32)
    y = y + bl_ref[...]
    if relu:
      y = jnp.maximum(y, 0.0)
    o_ref[...] = y

  return pl.pallas_call(
      body,
      grid=(N // BR,),
      in_specs=[
          pl.BlockSpec((NC, BR, HD), lambda i: (0, i, 0)),
          pl.BlockSpec((NC, BR, HD), lambda i: (0, i, 0)),
          pl.BlockSpec((NC, BR, CW), lambda i: (0, i, 0)),
          pl.BlockSpec((BR, D), lambda i: (i, 0)),
          pl.BlockSpec((D, D), lambda i: (0, 0)),
          pl.BlockSpec((1, D), lambda i: (0, 0)),
          pl.BlockSpec((D, D), lambda i: (0, 0)),
      ],
      out_specs=pl.BlockSpec((BR, D), lambda i: (i, 0)),
      out_shape=jax.ShapeDtypeStruct((N, D), jnp.float32),
  )(pa, pb, cnt, x, wl_t, bl, wr_t)


def kernel(x, edge_index, W1l, b1l, W1r, W2l, b2l, W2r):
  ei = edge_index.astype(jnp.int32)
  src_idx = ei[0].reshape(NW, NCHUNK, CHUNK)
  dst_idx = ei[1].reshape(NW, NCHUNK, CHUNK)
  z64 = jnp.zeros((STG, HD), jnp.float32)
  z16 = jnp.zeros((STG, CW), jnp.float32)
  ones = jnp.ones((CHUNK, CW), jnp.float32)

  xa = x[:, :HD]
  xb = x[:, HD:]
  pa1, pb1, cnt = _sc_agg_count(xa, xb, src_idx, dst_idx, z64, z16, ones)
  h = _combine(pa1, pb1, cnt, x, W1l.T, b1l.reshape(1, D), W1r.T, relu=True)
  pa2, pb2 = _sc_agg(h[:, :HD], h[:, HD:], src_idx, dst_idx, z64)
  out = _combine(pa2, pb2, cnt, h, W2l.T, b2l.reshape(1, D), W2r.T, relu=False)
  return out
